# baseline (speedup 1.0000x reference)
import jax
import jax.numpy as jnp
from jax.experimental import pallas as pl
from jax.experimental.pallas import tpu as pltpu

_B = 1024
_F = 4 * 64 * 64
_NT = 1000
_HB = 1024


def _body(t_ref, a_ref, b_ref, xa_ref, xb_ref, na_ref, nb_ref, o_ref,
          av_ref, bv_ref):
    @pl.when(pl.program_id(0) == 0)
    def _gather():
        t_row = t_ref[...].reshape(1, _B)
        iota = jax.lax.broadcasted_iota(jnp.int32, (_NT, _B), 0)
        oh = (iota == t_row).astype(jnp.float32)
        a_row = a_ref[...].reshape(1, _NT)
        b_row = b_ref[...].reshape(1, _NT)
        av_ref[...] = jax.lax.dot(a_row, oh,
                                  precision=jax.lax.Precision.HIGHEST,
                                  preferred_element_type=jnp.float32)
        bv_ref[...] = jax.lax.dot(b_row, oh,
                                  precision=jax.lax.Precision.HIGHEST,
                                  preferred_element_type=jnp.float32)

    av = av_ref[...]
    bv = bv_ref[...]
    o_ref[0:_HB, :] = av * xa_ref[...] + bv * na_ref[...]
    o_ref[_HB:2 * _HB, :] = av * xb_ref[...] + bv * nb_ref[...]


def kernel(x_0, timestep, noise, a, b):
    x2 = x_0.transpose(1, 2, 3, 0).reshape(_F, _B)
    n2 = noise.transpose(1, 2, 3, 0).reshape(_F, _B)

    grid = (_F // (2 * _HB),)
    out = pl.pallas_call(
        _body,
        grid=grid,
        in_specs=[
            pl.BlockSpec((_B,), lambda i: (0,)),
            pl.BlockSpec((_NT,), lambda i: (0,)),
            pl.BlockSpec((_NT,), lambda i: (0,)),
            pl.BlockSpec((_HB, _B), lambda i: (2 * i, 0)),
            pl.BlockSpec((_HB, _B), lambda i: (2 * i + 1, 0)),
            pl.BlockSpec((_HB, _B), lambda i: (2 * i, 0)),
            pl.BlockSpec((_HB, _B), lambda i: (2 * i + 1, 0)),
        ],
        out_specs=pl.BlockSpec((2 * _HB, _B), lambda i: (i, 0)),
        out_shape=jax.ShapeDtypeStruct((_F, _B), jnp.float32),
        scratch_shapes=[
            pltpu.VMEM((1, _B), jnp.float32),
            pltpu.VMEM((1, _B), jnp.float32),
        ],
        compiler_params=pltpu.CompilerParams(
            dimension_semantics=("arbitrary",),
        ),
    )(timestep, a, b, x2, x2, n2, n2)
    return out.reshape(4, 64, 64, _B).transpose(3, 0, 1, 2)


# speedup vs baseline: 1.0388x; 1.0388x over previous
import jax
import jax.numpy as jnp
from jax.experimental import pallas as pl
from jax.experimental.pallas import tpu as pltpu

_B = 1024
_F = 4 * 64 * 64
_NT = 1000
_HB = 1024


def _body(t_ref, a_ref, b_ref, xa_ref, xb_ref, na_ref, nb_ref, o_ref,
          av_ref, bv_ref):
    @pl.when(pl.program_id(0) == 0)
    def _gather():
        t_row = t_ref[...].reshape(1, _B)
        iota = jax.lax.broadcasted_iota(jnp.int32, (_NT, _B), 0)
        oh = (iota == t_row).astype(jnp.float32)
        a_row = a_ref[...].reshape(1, _NT)
        b_row = b_ref[...].reshape(1, _NT)

        def _sel(row):
            hi = row.astype(jnp.bfloat16).astype(jnp.float32)
            lo = row - hi
            return (jax.lax.dot(hi, oh, preferred_element_type=jnp.float32)
                    + jax.lax.dot(lo, oh, preferred_element_type=jnp.float32))

        av_ref[...] = _sel(a_row)
        bv_ref[...] = _sel(b_row)

    av = av_ref[...]
    bv = bv_ref[...]
    o_ref[0:_HB, :] = av * xa_ref[...] + bv * na_ref[...]
    o_ref[_HB:2 * _HB, :] = av * xb_ref[...] + bv * nb_ref[...]


def kernel(x_0, timestep, noise, a, b):
    x2 = x_0.transpose(1, 2, 3, 0).reshape(_F, _B)
    n2 = noise.transpose(1, 2, 3, 0).reshape(_F, _B)

    grid = (_F // (2 * _HB),)
    out = pl.pallas_call(
        _body,
        grid=grid,
        in_specs=[
            pl.BlockSpec((_B,), lambda i: (0,)),
            pl.BlockSpec((_NT,), lambda i: (0,)),
            pl.BlockSpec((_NT,), lambda i: (0,)),
            pl.BlockSpec((_HB, _B), lambda i: (2 * i, 0)),
            pl.BlockSpec((_HB, _B), lambda i: (2 * i + 1, 0)),
            pl.BlockSpec((_HB, _B), lambda i: (2 * i, 0)),
            pl.BlockSpec((_HB, _B), lambda i: (2 * i + 1, 0)),
        ],
        out_specs=pl.BlockSpec((2 * _HB, _B), lambda i: (i, 0)),
        out_shape=jax.ShapeDtypeStruct((_F, _B), jnp.float32),
        scratch_shapes=[
            pltpu.VMEM((1, _B), jnp.float32),
            pltpu.VMEM((1, _B), jnp.float32),
        ],
        compiler_params=pltpu.CompilerParams(
            dimension_semantics=("arbitrary",),
        ),
    )(timestep, a, b, x2, x2, n2, n2)
    return out.reshape(4, 64, 64, _B).transpose(3, 0, 1, 2)
